# baseline (speedup 1.0000x reference)
import functools

import jax
import jax.numpy as jnp
from jax.experimental import pallas as pl
from jax.experimental.pallas import tpu as pltpu

FUTURE_LEN = 60
DIM = 128
NUM_MODES = 6
NUM_EXPERTS = 5
TOP_K = 2
SHARED_W = 0.3
LB_W = 0.01

TILE = 256
MPAD = 8


def _gelu(x):
    return 0.5 * x * (1.0 + jax.lax.erf(x * 0.7071067811865476))


def _mlp(x, tw1, tb1, tw2, tb2, tw3, tb3, sw1, sb1, sw2, sb2, sw3, sb3):
    h = _gelu(jnp.dot(x, tw1, preferred_element_type=jnp.float32) + tb1)
    h = _gelu(jnp.dot(h, tw2, preferred_element_type=jnp.float32) + tb2)
    traj = jnp.dot(h, tw3, preferred_element_type=jnp.float32) + tb3
    g = _gelu(jnp.dot(x, sw1, preferred_element_type=jnp.float32) + sb1)
    g = _gelu(jnp.dot(g, sw2, preferred_element_type=jnp.float32) + sb2)
    score = jnp.dot(g, sw3, preferred_element_type=jnp.float32) + sb3
    return traj, score


def _body(n_tokens, n_steps,
          x_ref, rw1, rb1, rw2, rb2, rw3, rb3,
          stw1, stb1, stw2, stb2, stw3, stb3,
          ssw1, ssb1, ssw2, ssb2, ssw3, ssb3,
          utw1, utb1, utw2, utb2, utw3, utb3,
          usw1, usb1, usw2, usb2, usw3, usb3,
          ft_ref, fs_ref, rl_ref, tki_ref, aux_ref, et_ref, es_ref,
          acc_ref):
    i = pl.program_id(0)
    x3 = x_ref[...]
    bb = x3.shape[0]
    d = x3.shape[2]
    x8 = jnp.concatenate(
        [x3, jnp.zeros((bb, MPAD - NUM_MODES, d), jnp.float32)], axis=1)
    t = bb * MPAD
    x = x8.reshape(t, d)

    h = _gelu(jnp.dot(x, rw1[...], preferred_element_type=jnp.float32) + rb1[...])
    h = _gelu(jnp.dot(h, rw2[...], preferred_element_type=jnp.float32) + rb2[...])
    logits = jnp.dot(h, rw3[...], preferred_element_type=jnp.float32) + rb3[...]
    rl_ref[...] = logits.reshape(bb, MPAD, NUM_EXPERTS)[:, :NUM_MODES, :]

    v1 = jnp.max(logits, axis=1, keepdims=True)
    ex = jnp.exp(logits - v1)
    probs = ex / jnp.sum(ex, axis=1, keepdims=True)
    live = (jax.lax.broadcasted_iota(jnp.int32, (t, 1), 0) % MPAD) < NUM_MODES
    psum = jnp.sum(jnp.where(live, probs, 0.0), axis=0, keepdims=True)

    @pl.when(i == 0)
    def _init():
        acc_ref[...] = psum

    @pl.when(i > 0)
    def _acc():
        acc_ref[...] = acc_ref[...] + psum

    cols = jax.lax.broadcasted_iota(jnp.int32, (t, NUM_EXPERTS), 1)
    idx1 = jnp.min(jnp.where(logits == v1, cols, NUM_EXPERTS),
                   axis=1, keepdims=True)
    logits2 = jnp.where(cols == idx1, -jnp.inf, logits)
    v2 = jnp.max(logits2, axis=1, keepdims=True)
    idx2 = jnp.min(jnp.where(logits2 == v2, cols, NUM_EXPERTS),
                   axis=1, keepdims=True)
    tki_ref[...] = jnp.concatenate([idx1, idx2], axis=1).astype(
        jnp.int32).reshape(bb, MPAD, TOP_K)[:, :NUM_MODES, :]

    e2 = jnp.exp(v2 - v1)
    denom = 1.0 + e2
    p1 = 1.0 / denom
    p2 = e2 / denom
    wmat = p1 * (cols == idx1).astype(jnp.float32) \
        + p2 * (cols == idx2).astype(jnp.float32)

    straj, sscore = _mlp(x, stw1[...], stb1[...], stw2[...], stb2[...],
                         stw3[...], stb3[...], ssw1[...], ssb1[...],
                         ssw2[...], ssb2[...], ssw3[...], ssb3[...])

    f2 = FUTURE_LEN * 2
    ut = jnp.zeros((t, f2), jnp.float32)
    us = jnp.zeros((t, 1), jnp.float32)
    for e in range(NUM_EXPERTS):
        traj_e, score_e = _mlp(
            x, utw1[e], utb1[e:e + 1, :], utw2[e], utb2[e:e + 1, :],
            utw3[e], utb3[e:e + 1, :], usw1[e], usb1[e:e + 1, :],
            usw2[e], usb2[e:e + 1, :], usw3[e], usb3[e:e + 1, :])
        et_ref[e] = traj_e.reshape(bb, MPAD, f2)[:, :NUM_MODES, :]
        es_ref[e] = score_e
        w = wmat[:, e:e + 1]
        ut = ut + traj_e * w
        us = us + score_e * w

    ft = SHARED_W * straj + (1.0 - SHARED_W) * ut
    fs = SHARED_W * sscore + (1.0 - SHARED_W) * us
    ft_ref[...] = ft.reshape(bb, MPAD, f2)[:, :NUM_MODES, :]
    fs_ref[...] = fs

    @pl.when(i == n_steps - 1)
    def _aux():
        avg = acc_ref[...] / n_tokens
        entropy = -jnp.sum(avg * jnp.log(avg + 1e-08), axis=1, keepdims=True)
        lb = -entropy * LB_W
        uniform = 1.0 / NUM_EXPERTS
        l2 = jnp.mean((avg - uniform) ** 2, axis=1, keepdims=True)
        aux_ref[...] = lb + 0.01 * l2


def kernel(mode_features, params):
    b, m, d = mode_features.shape
    n = b * m
    p = params

    def b2(v):
        return v.reshape(1, -1)

    weights = [
        p['rw1'], b2(p['rb1']), p['rw2'], b2(p['rb2']), p['rw3'], b2(p['rb3']),
        p['s_tw1'], b2(p['s_tb1']), p['s_tw2'], b2(p['s_tb2']),
        p['s_tw3'], b2(p['s_tb3']),
        p['s_sw1'], b2(p['s_sb1']), p['s_sw2'], b2(p['s_sb2']),
        p['s_sw3'], b2(p['s_sb3']),
        p['u_tw1'], p['u_tb1'], p['u_tw2'], p['u_tb2'],
        p['u_tw3'], p['u_tb3'],
        p['u_sw1'], p['u_sb1'], p['u_sw2'], p['u_sb2'],
        p['u_sw3'], p['u_sb3'],
    ]

    n_steps = b // TILE
    tp = TILE * MPAD
    np_ = b * MPAD
    full = lambda a: pl.BlockSpec(a.shape, lambda i: (0,) * a.ndim)
    in_specs = [pl.BlockSpec((TILE, m, d), lambda i: (i, 0, 0))]
    in_specs += [full(w) for w in weights]

    f2 = FUTURE_LEN * 2
    out_shape = [
        jax.ShapeDtypeStruct((b, m, f2), jnp.float32),
        jax.ShapeDtypeStruct((np_, 1), jnp.float32),
        jax.ShapeDtypeStruct((b, m, NUM_EXPERTS), jnp.float32),
        jax.ShapeDtypeStruct((b, m, TOP_K), jnp.int32),
        jax.ShapeDtypeStruct((1, 1), jnp.float32),
        jax.ShapeDtypeStruct((NUM_EXPERTS, b, m, f2), jnp.float32),
        jax.ShapeDtypeStruct((NUM_EXPERTS, np_, 1), jnp.float32),
    ]
    out_specs = [
        pl.BlockSpec((TILE, m, f2), lambda i: (i, 0, 0)),
        pl.BlockSpec((tp, 1), lambda i: (i, 0)),
        pl.BlockSpec((TILE, m, NUM_EXPERTS), lambda i: (i, 0, 0)),
        pl.BlockSpec((TILE, m, TOP_K), lambda i: (i, 0, 0)),
        pl.BlockSpec((1, 1), lambda i: (0, 0)),
        pl.BlockSpec((NUM_EXPERTS, TILE, m, f2), lambda i: (0, i, 0, 0)),
        pl.BlockSpec((NUM_EXPERTS, tp, 1), lambda i: (0, i, 0)),
    ]

    ft, fs, rl, tki, aux, et, es = pl.pallas_call(
        functools.partial(_body, float(n), n_steps),
        grid=(n_steps,),
        in_specs=in_specs,
        out_specs=out_specs,
        out_shape=out_shape,
        scratch_shapes=[pltpu.VMEM((1, NUM_EXPERTS), jnp.float32)],
    )(mode_features, *weights)

    final_traj = ft.reshape(b, m, FUTURE_LEN, 2)
    final_score = fs.reshape(b, MPAD)[:, :m]
    aux_loss = aux[0, 0]
    expert_trajs = et.reshape(NUM_EXPERTS, b, m, FUTURE_LEN, 2)
    expert_scores = es.reshape(NUM_EXPERTS, b, MPAD)[:, :, :m]
    return (final_traj, final_score, rl, tki,
            aux_loss, expert_trajs, expert_scores)


# speedup vs baseline: 1.0855x; 1.0855x over previous
import functools

import jax
import jax.numpy as jnp
from jax.experimental import pallas as pl
from jax.experimental.pallas import tpu as pltpu

FUTURE_LEN = 60
DIM = 128
NUM_MODES = 6
NUM_EXPERTS = 5
TOP_K = 2
SHARED_W = 0.3
LB_W = 0.01

TILE = 256
MPAD = 8


def _gelu(x):
    return 0.5 * x * (1.0 + jax.lax.erf(x * 0.7071067811865476))


def _mlp(x, tw1, tb1, tw2, tb2, tw3, tb3, sw1, sb1, sw2, sb2, sw3, sb3):
    h = _gelu(jnp.dot(x, tw1, preferred_element_type=jnp.float32) + tb1)
    h = _gelu(jnp.dot(h, tw2, preferred_element_type=jnp.float32) + tb2)
    traj = jnp.dot(h, tw3, preferred_element_type=jnp.float32) + tb3
    g = _gelu(jnp.dot(x, sw1, preferred_element_type=jnp.float32) + sb1)
    g = _gelu(jnp.dot(g, sw2, preferred_element_type=jnp.float32) + sb2)
    score = jnp.dot(g, sw3, preferred_element_type=jnp.float32) + sb3
    return traj, score


def _body(n_tokens, n_steps,
          x_ref, rw1, rb1, rw2, rb2, rw3, rb3,
          stw1, stb1, stw2, stb2, stw3, stb3,
          ssw1, ssb1, ssw2, ssb2, ssw3, ssb3,
          utw1, utb1, utw2, utb2, utw3, utb3,
          usw1, usb1, usw2, usb2, usw3, usb3,
          ft_ref, fs_ref, rl_ref, tki_ref, aux_ref, et_ref, es_ref,
          acc_ref):
    i = pl.program_id(0)
    x3 = x_ref[...]
    bb = x3.shape[0]
    d = x3.shape[2]
    x8 = jnp.concatenate(
        [x3, jnp.zeros((bb, MPAD - NUM_MODES, d), jnp.float32)], axis=1)
    t = bb * MPAD
    x = x8.reshape(t, d)

    h = _gelu(jnp.dot(x, rw1[...], preferred_element_type=jnp.float32) + rb1[...])
    h = _gelu(jnp.dot(h, rw2[...], preferred_element_type=jnp.float32) + rb2[...])
    logits = jnp.dot(h, rw3[...], preferred_element_type=jnp.float32) + rb3[...]
    rl_ref[...] = logits.reshape(bb, MPAD, NUM_EXPERTS)[:, :NUM_MODES, :]

    v1 = jnp.max(logits, axis=1, keepdims=True)
    ex = jnp.exp(logits - v1)
    probs = ex / jnp.sum(ex, axis=1, keepdims=True)
    live = (jax.lax.broadcasted_iota(jnp.int32, (t, 1), 0) % MPAD) < NUM_MODES
    psum = jnp.sum(jnp.where(live, probs, 0.0), axis=0, keepdims=True)

    @pl.when(i == 0)
    def _init():
        acc_ref[...] = psum

    @pl.when(i > 0)
    def _acc():
        acc_ref[...] = acc_ref[...] + psum

    cols = jax.lax.broadcasted_iota(jnp.int32, (t, NUM_EXPERTS), 1)
    idx1 = jnp.min(jnp.where(logits == v1, cols, NUM_EXPERTS),
                   axis=1, keepdims=True)
    logits2 = jnp.where(cols == idx1, -jnp.inf, logits)
    v2 = jnp.max(logits2, axis=1, keepdims=True)
    idx2 = jnp.min(jnp.where(logits2 == v2, cols, NUM_EXPERTS),
                   axis=1, keepdims=True)
    tki_ref[...] = jnp.concatenate([idx1, idx2], axis=1).astype(
        jnp.int32).reshape(bb, MPAD, TOP_K)[:, :NUM_MODES, :]

    e2 = jnp.exp(v2 - v1)
    denom = 1.0 + e2
    p1 = 1.0 / denom
    p2 = e2 / denom
    wmat = p1 * (cols == idx1).astype(jnp.float32) \
        + p2 * (cols == idx2).astype(jnp.float32)

    straj, sscore = _mlp(x, stw1[...], stb1[...], stw2[...], stb2[...],
                         stw3[...], stb3[...], ssw1[...], ssb1[...],
                         ssw2[...], ssb2[...], ssw3[...], ssb3[...])

    f2 = FUTURE_LEN * 2
    ut = jnp.zeros((t, f2), jnp.float32)
    us = jnp.zeros((t, 1), jnp.float32)
    for e in range(NUM_EXPERTS):
        traj_e, score_e = _mlp(
            x, utw1[e], utb1[e:e + 1, :], utw2[e], utb2[e:e + 1, :],
            utw3[e], utb3[e:e + 1, :], usw1[e], usb1[e:e + 1, :],
            usw2[e], usb2[e:e + 1, :], usw3[e], usb3[e:e + 1, :])
        et_ref[e] = traj_e.reshape(bb, MPAD, f2)[:, :NUM_MODES, :]
        es_ref[e] = score_e.reshape(bb, MPAD)[:, :NUM_MODES]
        w = wmat[:, e:e + 1]
        ut = ut + traj_e * w
        us = us + score_e * w

    ft = SHARED_W * straj + (1.0 - SHARED_W) * ut
    fs = SHARED_W * sscore + (1.0 - SHARED_W) * us
    ft_ref[...] = ft.reshape(bb, MPAD, f2)[:, :NUM_MODES, :]
    fs_ref[...] = fs.reshape(bb, MPAD)[:, :NUM_MODES]

    @pl.when(i == n_steps - 1)
    def _aux():
        avg = acc_ref[...] / n_tokens
        entropy = -jnp.sum(avg * jnp.log(avg + 1e-08), axis=1, keepdims=True)
        lb = -entropy * LB_W
        uniform = 1.0 / NUM_EXPERTS
        l2 = jnp.mean((avg - uniform) ** 2, axis=1, keepdims=True)
        aux_ref[...] = lb + 0.01 * l2


def kernel(mode_features, params):
    b, m, d = mode_features.shape
    n = b * m
    p = params

    def b2(v):
        return v.reshape(1, -1)

    weights = [
        p['rw1'], b2(p['rb1']), p['rw2'], b2(p['rb2']), p['rw3'], b2(p['rb3']),
        p['s_tw1'], b2(p['s_tb1']), p['s_tw2'], b2(p['s_tb2']),
        p['s_tw3'], b2(p['s_tb3']),
        p['s_sw1'], b2(p['s_sb1']), p['s_sw2'], b2(p['s_sb2']),
        p['s_sw3'], b2(p['s_sb3']),
        p['u_tw1'], p['u_tb1'], p['u_tw2'], p['u_tb2'],
        p['u_tw3'], p['u_tb3'],
        p['u_sw1'], p['u_sb1'], p['u_sw2'], p['u_sb2'],
        p['u_sw3'], p['u_sb3'],
    ]

    n_steps = b // TILE
    tp = TILE * MPAD
    np_ = b * MPAD
    full = lambda a: pl.BlockSpec(a.shape, lambda i: (0,) * a.ndim)
    in_specs = [pl.BlockSpec((TILE, m, d), lambda i: (i, 0, 0))]
    in_specs += [full(w) for w in weights]

    f2 = FUTURE_LEN * 2
    out_shape = [
        jax.ShapeDtypeStruct((b, m, f2), jnp.float32),
        jax.ShapeDtypeStruct((b, m), jnp.float32),
        jax.ShapeDtypeStruct((b, m, NUM_EXPERTS), jnp.float32),
        jax.ShapeDtypeStruct((b, m, TOP_K), jnp.int32),
        jax.ShapeDtypeStruct((1, 1), jnp.float32),
        jax.ShapeDtypeStruct((NUM_EXPERTS, b, m, f2), jnp.float32),
        jax.ShapeDtypeStruct((NUM_EXPERTS, b, m), jnp.float32),
    ]
    out_specs = [
        pl.BlockSpec((TILE, m, f2), lambda i: (i, 0, 0)),
        pl.BlockSpec((TILE, m), lambda i: (i, 0)),
        pl.BlockSpec((TILE, m, NUM_EXPERTS), lambda i: (i, 0, 0)),
        pl.BlockSpec((TILE, m, TOP_K), lambda i: (i, 0, 0)),
        pl.BlockSpec((1, 1), lambda i: (0, 0)),
        pl.BlockSpec((NUM_EXPERTS, TILE, m, f2), lambda i: (0, i, 0, 0)),
        pl.BlockSpec((NUM_EXPERTS, TILE, m), lambda i: (0, i, 0)),
    ]

    ft, fs, rl, tki, aux, et, es = pl.pallas_call(
        functools.partial(_body, float(n), n_steps),
        grid=(n_steps,),
        in_specs=in_specs,
        out_specs=out_specs,
        out_shape=out_shape,
        scratch_shapes=[pltpu.VMEM((1, NUM_EXPERTS), jnp.float32)],
    )(mode_features, *weights)

    final_traj = ft.reshape(b, m, FUTURE_LEN, 2)
    aux_loss = aux[0, 0]
    expert_trajs = et.reshape(NUM_EXPERTS, b, m, FUTURE_LEN, 2)
    return (final_traj, fs, rl, tki, aux_loss, expert_trajs, es)
